# baseline (speedup 1.0000x reference)
import functools

import jax
import jax.numpy as jnp
from jax import lax
from jax.experimental import pallas as pl
from jax.experimental.pallas import tpu as pltpu
from jax.experimental.pallas import tpu_sc as plsc

NC = 2
NS = 16
L = 16


def _xw_body(x_ref, w_ref, o_ref):
    o_ref[0] = jnp.dot(x_ref[...], w_ref[0], preferred_element_type=jnp.float32)


def _combine_body(p_ref, xws_ref, b_ref, o_ref):
    o_ref[...] = p_ref[0] + p_ref[1] + xws_ref[0] + b_ref[...]


def _make_deg_call(E, N, R):
    RN = R * N
    KB = 80
    NSUB = 25
    KG = NSUB * KB
    EA = E // (NC * NS)
    DT = 11
    DZ = RN // DT
    assert E % (NC * NS * KG) == 0 and RN % DT == 0
    assert DZ % KG == 0 and DZ % 8 == 0

    mesh = plsc.VectorSubcoreMesh(
        core_axis_name="c", subcore_axis_name="s",
        num_cores=NC, num_subcores=NS)

    @functools.partial(
        pl.kernel, mesh=mesh,
        compiler_params=pltpu.CompilerParams(needs_layout_passes=False),
        out_type=jax.ShapeDtypeStruct((NC * RN,), jnp.float32),
        scratch_types=[
            pltpu.VMEM((KG,), jnp.int32),
            pltpu.VMEM((KG,), jnp.int32),
            pltpu.VMEM((NSUB, KB), jnp.int32),
            pltpu.VMEM((KG,), jnp.float32),
            pltpu.VMEM((KB,), jnp.float32),
            pltpu.VMEM_SHARED((RN,), jnp.float32),
            pltpu.SemaphoreType.DMA,
            pltpu.SemaphoreType.DMA,
            pltpu.SemaphoreType.DMA,
        ],
    )
    def deg_call(src_hbm, et_hbm, out_hbm,
                 srcE, etE, gidx2d, zsrc, onesb, deg_sp, sg0, sg1, semA):
        c = lax.axis_index("c")
        s = lax.axis_index("s")
        wid = c * NS + s
        zeros = jnp.zeros((L,), jnp.float32)
        ones = jnp.ones((L,), jnp.float32)
        for k in range(KG // L):
            zsrc[pl.ds(k * L, L)] = zeros
        for k in range(KB // L):
            onesb[pl.ds(k * L, L)] = ones

        @pl.when(s < DT)
        def _zero_deg():
            for i in range(DZ // KG):
                pltpu.sync_copy(zsrc, deg_sp.at[pl.ds(s * DZ + i * KG, KG)])

        plsc.subcore_barrier()

        def group_a(gi, _):
            base = wid * EA + gi * KG
            e1 = pltpu.async_copy(src_hbm.at[pl.ds(base, KG)], srcE, sg0)
            e2 = pltpu.async_copy(et_hbm.at[pl.ds(base, KG)], etE, sg1)
            e1.wait()
            e2.wait()
            def idx_outer(jj, _):
                def inner(tt, _):
                    sl = pl.ds(jj * KB + tt * L, L)
                    gidx2d[jj, pl.ds(tt * L, L)] = etE[sl] * N + srcE[sl]
                    return 0
                lax.fori_loop(0, KB // L, inner, 0, unroll=KB // L)
                return 0
            lax.fori_loop(0, NSUB, idx_outer, 0)
            def scat(b, _):
                descs = [pltpu.async_copy(onesb,
                                          deg_sp.at[gidx2d.at[b * 5 + u]],
                                          semA, add=True)
                         for u in range(5)]
                for d in descs:
                    d.wait()
                return 0
            lax.fori_loop(0, NSUB // 5, scat, 0)
            return 0
        lax.fori_loop(0, EA // KG, group_a, 0)

        plsc.subcore_barrier()

        @pl.when(s < DT)
        def _copy_out():
            for i in range(DZ // KG):
                pltpu.sync_copy(deg_sp.at[pl.ds(s * DZ + i * KG, KG)], zsrc)
                pltpu.sync_copy(
                    zsrc, out_hbm.at[pl.ds(c * RN + s * DZ + i * KG, KG)])

    return deg_call


def _make_sc_call(E, N, R, D):
    RN = R * N
    NW = NC * NS
    EP = E // NW
    EA = E // NS
    KB = 80
    NSUB = 25
    KG = NSUB * KB
    ZR = 40
    CT = 10
    RPT = N // CT
    assert E % (NW * KG) == 0
    assert N % CT == 0 and RPT % ZR == 0 and RPT % 8 == 0
    assert D % L == 0 and KB % L == 0 and KB % 8 == 0

    mesh = plsc.VectorSubcoreMesh(
        core_axis_name="c", subcore_axis_name="s",
        num_cores=NC, num_subcores=NS)

    @functools.partial(
        pl.kernel, mesh=mesh,
        compiler_params=pltpu.CompilerParams(needs_layout_passes=False),
        out_type=jax.ShapeDtypeStruct((NC, N, D), jnp.float32),
        scratch_types=[
            pltpu.VMEM((KG,), jnp.int32),
            pltpu.VMEM((KG,), jnp.int32),
            pltpu.VMEM((KG,), jnp.int32),
            pltpu.VMEM((NSUB, KB), jnp.int32),
            pltpu.VMEM((NSUB, KB), jnp.int32),
            pltpu.VMEM((KG,), jnp.int32),
            pltpu.VMEM((KG,), jnp.float32),
            pltpu.VMEM((KG,), jnp.float32),
            pltpu.VMEM((KB, D), jnp.float32),
            pltpu.VMEM((KB, D), jnp.float32),
            pltpu.VMEM((ZR, D), jnp.float32),
            pltpu.VMEM_SHARED((N, D), jnp.float32),
            pltpu.SemaphoreType.DMA,
            pltpu.SemaphoreType.DMA,
            pltpu.SemaphoreType.DMA,
            pltpu.SemaphoreType.DMA,
            pltpu.SemaphoreType.DMA,
        ],
    )
    def sc_call(src_hbm, dst_hbm, et_hbm, xw_hbm, deg0_hbm, deg1_hbm, out_hbm,
                srcE, dstE, etE, sidx2d, gidx2d, fridx, dvals, dvals2,
                rows0, rows1, zbuf, acc,
                semD, sg0, sg1, ss0, ss1):
        c = lax.axis_index("c")
        s = lax.axis_index("s")
        wid = c * NS + s
        zeros = jnp.zeros((L,), jnp.float32)

        for rr in range(ZR):
            for k in range(D // L):
                zbuf[rr, pl.ds(k * L, L)] = zeros

        @pl.when(s < CT)
        def _zero_acc():
            for i in range(RPT // ZR):
                pltpu.sync_copy(zbuf, acc.at[pl.ds(s * RPT + i * ZR, ZR)])

        plsc.subcore_barrier()

        def group_b(g, _):
            base = wid * EP + g * KG
            e1 = pltpu.async_copy(src_hbm.at[pl.ds(base, KG)], srcE, sg0)
            e2 = pltpu.async_copy(dst_hbm.at[pl.ds(base, KG)], dstE, sg1)
            e3 = pltpu.async_copy(et_hbm.at[pl.ds(base, KG)], etE, semD)
            e1.wait()
            e2.wait()
            e3.wait()
            def idx_outer(jj, _):
                def idx_loop(tt, _):
                    sl = pl.ds(jj * KB + tt * L, L)
                    sl2 = pl.ds(tt * L, L)
                    s16 = srcE[sl]
                    e16 = etE[sl]
                    sidx2d[jj, sl2] = s16
                    gidx2d[jj, sl2] = e16 * N + dstE[sl]
                    fridx[sl] = e16 * N + s16
                    return 0
                lax.fori_loop(0, KB // L, idx_loop, 0, unroll=KB // L)
                return 0
            lax.fori_loop(0, NSUB, idx_outer, 0)

            pltpu.async_copy(xw_hbm.at[gidx2d.at[0]], rows0, sg0)
            pltpu.async_copy(xw_hbm.at[gidx2d.at[1]], rows1, sg1)

            def dgat(b, _):
                dd = [pltpu.async_copy(
                        deg0_hbm.at[fridx.at[pl.ds((b * 5 + u) * KB, KB)]],
                        dvals.at[pl.ds((b * 5 + u) * KB, KB)], semD)
                      for u in range(5)]
                dd += [pltpu.async_copy(
                        deg1_hbm.at[fridx.at[pl.ds((b * 5 + u) * KB, KB)]],
                        dvals2.at[pl.ds((b * 5 + u) * KB, KB)], semD)
                       for u in range(5)]
                for d in dd:
                    d.wait()
                return 0
            lax.fori_loop(0, NSUB // 5, dgat, 0)
            def inv_loop(t, _):
                sl = pl.ds(t * L, L)
                dvals[sl] = 1.0 / (dvals[sl] + dvals2[sl])
                return 0
            lax.fori_loop(0, KG // L, inv_loop, 0, unroll=5)

            def do_sub(jj, buf, sgk, ssk, refill=True):
                pltpu.make_async_copy(xw_hbm.at[gidx2d.at[jj]],
                                      buf, sgk).wait()
                def scale(i, _):
                    bv = plsc.load_gather(
                        dvals, [jnp.full((L,), jj * KB, jnp.int32) + i])
                    for sub in range(D // L):
                        sl = pl.ds(sub * L, L)
                        buf[i, sl] = buf[i, sl] * bv
                    return 0
                lax.fori_loop(0, KB, scale, 0, unroll=2)
                pltpu.async_copy(buf, acc.at[sidx2d.at[jj]],
                                 ssk, add=True).wait()
                if refill:
                    @pl.when(jj + 2 < NSUB)
                    def _refill():
                        pltpu.async_copy(xw_hbm.at[gidx2d.at[jj + 2]],
                                         buf, sgk)

            def pair(p, _):
                do_sub(p * 2, rows0, sg0, ss0)
                do_sub(p * 2 + 1, rows1, sg1, ss1)
                return 0
            lax.fori_loop(0, NSUB // 2, pair, 0)
            do_sub(NSUB - 1, rows0, sg0, ss0, refill=False)
            return 0
        lax.fori_loop(0, EP // KG, group_b, 0)

        plsc.subcore_barrier()

        @pl.when(s < CT)
        def _copy_out():
            pltpu.sync_copy(acc.at[pl.ds(s * RPT, RPT)],
                            out_hbm.at[c, pl.ds(s * RPT, RPT)])

    return sc_call


def kernel(x, r, edge_index, edge_type, weights, bias):
    N, D_IN = x.shape
    R, _, D_OUT = weights.shape
    E = edge_type.shape[0]
    BN = 2000

    deg_call = _make_deg_call(E, N, R)
    degs = deg_call(edge_index[0], edge_type)

    xw = pl.pallas_call(
        _xw_body,
        grid=(R, N // BN),
        in_specs=[
            pl.BlockSpec((BN, D_IN), lambda rr, nb: (nb, 0)),
            pl.BlockSpec((1, D_IN, D_OUT), lambda rr, nb: (rr, 0, 0)),
        ],
        out_specs=pl.BlockSpec((1, BN, D_OUT), lambda rr, nb: (rr, nb, 0)),
        out_shape=jax.ShapeDtypeStruct((R, N, D_OUT), jnp.float32),
    )(x, weights)

    sc_call = _make_sc_call(E, N, R, D_OUT)
    partials = sc_call(edge_index[0], edge_index[1], edge_type,
                       xw.reshape(R * N, D_OUT),
                       degs[:R * N], degs[R * N:])

    out = pl.pallas_call(
        _combine_body,
        grid=(N // BN,),
        in_specs=[
            pl.BlockSpec((NC, BN, D_OUT), lambda nb: (0, nb, 0)),
            pl.BlockSpec((1, BN, D_OUT), lambda nb: (R - 1, nb, 0)),
            pl.BlockSpec((1, D_OUT), lambda nb: (0, 0)),
        ],
        out_specs=pl.BlockSpec((BN, D_OUT), lambda nb: (nb, 0)),
        out_shape=jax.ShapeDtypeStruct((N, D_OUT), jnp.float32),
    )(partials, xw, bias.reshape(1, D_OUT))

    return (out, r)


# speedup vs baseline: 1.0663x; 1.0663x over previous
import functools

import jax
import jax.numpy as jnp
from jax import lax
from jax.experimental import pallas as pl
from jax.experimental.pallas import tpu as pltpu
from jax.experimental.pallas import tpu_sc as plsc

NC = 2
NS = 16
L = 16


def _xw_body(x_ref, w_ref, o_ref):
    o_ref[0] = jnp.dot(x_ref[...], w_ref[0], preferred_element_type=jnp.float32)


def _combine_body(p_ref, xws_ref, b_ref, o_ref):
    o_ref[...] = p_ref[0] + p_ref[1] + xws_ref[0] + b_ref[...]


def _make_deg_call(E, N, R):
    RN = R * N
    KB = 80
    NSUB = 25
    KG = NSUB * KB
    EA = E // NS
    DT = 11
    DZ = RN // DT
    assert E % (NS * KG) == 0 and RN % DT == 0
    assert DZ % KG == 0 and DZ % 8 == 0

    mesh = plsc.VectorSubcoreMesh(
        core_axis_name="c", subcore_axis_name="s",
        num_cores=NC, num_subcores=NS)

    @functools.partial(
        pl.kernel, mesh=mesh,
        compiler_params=pltpu.CompilerParams(needs_layout_passes=False),
        out_type=jax.ShapeDtypeStruct((NC * RN,), jnp.float32),
        scratch_types=[
            pltpu.VMEM((KG,), jnp.int32),
            pltpu.VMEM((KG,), jnp.int32),
            pltpu.VMEM((NSUB, KB), jnp.int32),
            pltpu.VMEM((KG,), jnp.float32),
            pltpu.VMEM((KB,), jnp.float32),
            pltpu.VMEM_SHARED((RN,), jnp.float32),
            pltpu.SemaphoreType.DMA,
            pltpu.SemaphoreType.DMA,
            pltpu.SemaphoreType.DMA,
        ],
    )
    def deg_call(src_hbm, et_hbm, out_hbm,
                 srcE, etE, gidx2d, zsrc, onesb, deg_sp, sg0, sg1, semA):
        c = lax.axis_index("c")
        s = lax.axis_index("s")
        wid = c * NS + s
        zeros = jnp.zeros((L,), jnp.float32)
        ones = jnp.ones((L,), jnp.float32)
        for k in range(KG // L):
            zsrc[pl.ds(k * L, L)] = zeros
        for k in range(KB // L):
            onesb[pl.ds(k * L, L)] = ones

        @pl.when(s < DT)
        def _zero_deg():
            for i in range(DZ // KG):
                pltpu.sync_copy(zsrc, deg_sp.at[pl.ds(s * DZ + i * KG, KG)])

        plsc.subcore_barrier()

        def group_a(gi, _):
            base = s * EA + gi * KG
            e1 = pltpu.async_copy(src_hbm.at[pl.ds(base, KG)], srcE, sg0)
            e2 = pltpu.async_copy(et_hbm.at[pl.ds(base, KG)], etE, sg1)
            e1.wait()
            e2.wait()
            def idx_outer(jj, _):
                def inner(tt, _):
                    sl = pl.ds(jj * KB + tt * L, L)
                    gidx2d[jj, pl.ds(tt * L, L)] = etE[sl] * N + srcE[sl]
                    return 0
                lax.fori_loop(0, KB // L, inner, 0, unroll=KB // L)
                return 0
            lax.fori_loop(0, NSUB, idx_outer, 0)
            def scat(b, _):
                descs = [pltpu.async_copy(onesb,
                                          deg_sp.at[gidx2d.at[b * 5 + u]],
                                          semA, add=True)
                         for u in range(5)]
                for d in descs:
                    d.wait()
                return 0
            lax.fori_loop(0, NSUB // 5, scat, 0)
            return 0
        lax.fori_loop(0, EA // KG, group_a, 0)

        plsc.subcore_barrier()

        @pl.when(s < DT)
        def _copy_out():
            for i in range(DZ // KG):
                pltpu.sync_copy(deg_sp.at[pl.ds(s * DZ + i * KG, KG)], zsrc)
                pltpu.sync_copy(
                    zsrc, out_hbm.at[pl.ds(c * RN + s * DZ + i * KG, KG)])

    return deg_call


def _make_sc_call(E, N, R, D):
    RN = R * N
    NW = NC * NS
    EP = E // NW
    EA = E // NS
    KB = 80
    NSUB = 25
    KG = NSUB * KB
    ZR = 40
    CT = 10
    RPT = N // CT
    assert E % (NW * KG) == 0
    assert N % CT == 0 and RPT % ZR == 0 and RPT % 8 == 0
    assert D % L == 0 and KB % L == 0 and KB % 8 == 0

    mesh = plsc.VectorSubcoreMesh(
        core_axis_name="c", subcore_axis_name="s",
        num_cores=NC, num_subcores=NS)

    @functools.partial(
        pl.kernel, mesh=mesh,
        compiler_params=pltpu.CompilerParams(needs_layout_passes=False),
        out_type=jax.ShapeDtypeStruct((NC, N, D), jnp.float32),
        scratch_types=[
            pltpu.VMEM((KG,), jnp.int32),
            pltpu.VMEM((KG,), jnp.int32),
            pltpu.VMEM((KG,), jnp.int32),
            pltpu.VMEM((NSUB, KB), jnp.int32),
            pltpu.VMEM((NSUB, KB), jnp.int32),
            pltpu.VMEM((KG,), jnp.int32),
            pltpu.VMEM((KG,), jnp.float32),
            pltpu.VMEM((KB, D), jnp.float32),
            pltpu.VMEM((KB, D), jnp.float32),
            pltpu.VMEM((ZR, D), jnp.float32),
            pltpu.VMEM_SHARED((N, D), jnp.float32),
            pltpu.SemaphoreType.DMA,
            pltpu.SemaphoreType.DMA,
            pltpu.SemaphoreType.DMA,
            pltpu.SemaphoreType.DMA,
            pltpu.SemaphoreType.DMA,
        ],
    )
    def sc_call(src_hbm, dst_hbm, et_hbm, xw_hbm, deg_hbm, out_hbm,
                srcE, dstE, etE, sidx2d, gidx2d, fridx, dvals,
                rows0, rows1, zbuf, acc,
                semD, sg0, sg1, ss0, ss1):
        c = lax.axis_index("c")
        s = lax.axis_index("s")
        wid = c * NS + s
        zeros = jnp.zeros((L,), jnp.float32)

        for rr in range(ZR):
            for k in range(D // L):
                zbuf[rr, pl.ds(k * L, L)] = zeros

        @pl.when(s < CT)
        def _zero_acc():
            for i in range(RPT // ZR):
                pltpu.sync_copy(zbuf, acc.at[pl.ds(s * RPT + i * ZR, ZR)])

        plsc.subcore_barrier()

        def group_b(g, _):
            base = wid * EP + g * KG
            e1 = pltpu.async_copy(src_hbm.at[pl.ds(base, KG)], srcE, sg0)
            e2 = pltpu.async_copy(dst_hbm.at[pl.ds(base, KG)], dstE, sg1)
            e3 = pltpu.async_copy(et_hbm.at[pl.ds(base, KG)], etE, semD)
            e1.wait()
            e2.wait()
            e3.wait()
            def idx_outer(jj, _):
                def idx_loop(tt, _):
                    sl = pl.ds(jj * KB + tt * L, L)
                    sl2 = pl.ds(tt * L, L)
                    s16 = srcE[sl]
                    e16 = etE[sl]
                    sidx2d[jj, sl2] = s16
                    gidx2d[jj, sl2] = e16 * N + dstE[sl]
                    fridx[sl] = c * RN + e16 * N + s16
                    return 0
                lax.fori_loop(0, KB // L, idx_loop, 0, unroll=KB // L)
                return 0
            lax.fori_loop(0, NSUB, idx_outer, 0)

            pltpu.async_copy(xw_hbm.at[gidx2d.at[0]], rows0, sg0)
            pltpu.async_copy(xw_hbm.at[gidx2d.at[1]], rows1, sg1)

            def dgat(b, _):
                dd = [pltpu.async_copy(
                        deg_hbm.at[fridx.at[pl.ds((b * 5 + u) * KB, KB)]],
                        dvals.at[pl.ds((b * 5 + u) * KB, KB)], semD)
                      for u in range(5)]
                for d in dd:
                    d.wait()
                return 0
            lax.fori_loop(0, NSUB // 5, dgat, 0)
            def inv_loop(t, _):
                sl = pl.ds(t * L, L)
                dvals[sl] = 1.0 / dvals[sl]
                return 0
            lax.fori_loop(0, KG // L, inv_loop, 0, unroll=5)

            def do_sub(jj, buf, sgk, ssk, refill=True):
                pltpu.make_async_copy(xw_hbm.at[gidx2d.at[jj]],
                                      buf, sgk).wait()
                def scale(i, _):
                    bv = plsc.load_gather(
                        dvals, [jnp.full((L,), jj * KB, jnp.int32) + i])
                    for sub in range(D // L):
                        sl = pl.ds(sub * L, L)
                        buf[i, sl] = buf[i, sl] * bv
                    return 0
                lax.fori_loop(0, KB, scale, 0, unroll=2)
                pltpu.async_copy(buf, acc.at[sidx2d.at[jj]],
                                 ssk, add=True).wait()
                if refill:
                    @pl.when(jj + 2 < NSUB)
                    def _refill():
                        pltpu.async_copy(xw_hbm.at[gidx2d.at[jj + 2]],
                                         buf, sgk)

            def pair(p, _):
                do_sub(p * 2, rows0, sg0, ss0)
                do_sub(p * 2 + 1, rows1, sg1, ss1)
                return 0
            lax.fori_loop(0, NSUB // 2, pair, 0)
            do_sub(NSUB - 1, rows0, sg0, ss0, refill=False)
            return 0
        lax.fori_loop(0, EP // KG, group_b, 0)

        plsc.subcore_barrier()

        @pl.when(s < CT)
        def _copy_out():
            pltpu.sync_copy(acc.at[pl.ds(s * RPT, RPT)],
                            out_hbm.at[c, pl.ds(s * RPT, RPT)])

    return sc_call


def kernel(x, r, edge_index, edge_type, weights, bias):
    N, D_IN = x.shape
    R, _, D_OUT = weights.shape
    E = edge_type.shape[0]
    BN = 2000

    deg_call = _make_deg_call(E, N, R)
    degs = deg_call(edge_index[0], edge_type)

    xw = pl.pallas_call(
        _xw_body,
        grid=(R, N // BN),
        in_specs=[
            pl.BlockSpec((BN, D_IN), lambda rr, nb: (nb, 0)),
            pl.BlockSpec((1, D_IN, D_OUT), lambda rr, nb: (rr, 0, 0)),
        ],
        out_specs=pl.BlockSpec((1, BN, D_OUT), lambda rr, nb: (rr, nb, 0)),
        out_shape=jax.ShapeDtypeStruct((R, N, D_OUT), jnp.float32),
    )(x, weights)

    sc_call = _make_sc_call(E, N, R, D_OUT)
    partials = sc_call(edge_index[0], edge_index[1], edge_type,
                       xw.reshape(R * N, D_OUT), degs)

    out = pl.pallas_call(
        _combine_body,
        grid=(N // BN,),
        in_specs=[
            pl.BlockSpec((NC, BN, D_OUT), lambda nb: (0, nb, 0)),
            pl.BlockSpec((1, BN, D_OUT), lambda nb: (R - 1, nb, 0)),
            pl.BlockSpec((1, D_OUT), lambda nb: (0, 0)),
        ],
        out_specs=pl.BlockSpec((BN, D_OUT), lambda nb: (nb, 0)),
        out_shape=jax.ShapeDtypeStruct((N, D_OUT), jnp.float32),
    )(partials, xw, bias.reshape(1, D_OUT))

    return (out, r)
